# baseline (speedup 1.0000x reference)
import jax
import jax.numpy as jnp
from jax.experimental import pallas as pl
from jax.experimental.pallas import tpu as pltpu

_N_CHUNKS = 8


def _dma_copy(table_ref, out_ref, sems):
    rows = table_ref.shape[0] // _N_CHUNKS
    for i in range(_N_CHUNKS):
        pltpu.make_async_copy(
            table_ref.at[pl.ds(i * rows, rows)],
            out_ref.at[pl.ds(i * rows, rows)],
            sems.at[i],
        ).start()
    for i in range(_N_CHUNKS):
        pltpu.make_async_copy(
            table_ref.at[pl.ds(i * rows, rows)],
            out_ref.at[pl.ds(i * rows, rows)],
            sems.at[i],
        ).wait()


def kernel(x, table):
    seq_len = x.shape[1]
    d_model = table.shape[1]
    out = pl.pallas_call(
        _dma_copy,
        in_specs=[pl.BlockSpec(memory_space=pl.ANY)],
        out_specs=pl.BlockSpec(memory_space=pl.ANY),
        out_shape=jax.ShapeDtypeStruct((seq_len, d_model), table.dtype),
        scratch_shapes=[pltpu.SemaphoreType.DMA((_N_CHUNKS,))],
    )(table)
    return out[None, :, :]


# speedup vs baseline: 40.9050x; 40.9050x over previous
import jax
import jax.numpy as jnp
from jax.experimental import pallas as pl
from jax.experimental.pallas import tpu as pltpu

_BLOCK_ROWS = 512


def _copy_block(table_ref, out_ref):
    out_ref[...] = table_ref[...]


def kernel(x, table):
    seq_len = x.shape[1]
    d_model = table.shape[1]
    out = pl.pallas_call(
        _copy_block,
        grid=(seq_len // _BLOCK_ROWS,),
        in_specs=[pl.BlockSpec((_BLOCK_ROWS, d_model), lambda i: (i, 0))],
        out_specs=pl.BlockSpec((_BLOCK_ROWS, d_model), lambda i: (i, 0)),
        out_shape=jax.ShapeDtypeStruct((seq_len, d_model), table.dtype),
        compiler_params=pltpu.CompilerParams(
            dimension_semantics=("parallel",),
        ),
    )(table)
    return out[None, :, :]


# speedup vs baseline: 46.9368x; 1.1475x over previous
import jax
import jax.numpy as jnp
from jax.experimental import pallas as pl
from jax.experimental.pallas import tpu as pltpu

_N_CHUNKS = 16


def _dma_copy(table_ref, out_ref, scratch, in_sems, out_sems):
    rows = table_ref.shape[0] // _N_CHUNKS

    def in_copy(i):
        return pltpu.make_async_copy(
            table_ref.at[pl.ds(i * rows, rows)],
            scratch.at[pl.ds(i * rows, rows)],
            in_sems.at[i],
        )

    def out_copy(i):
        return pltpu.make_async_copy(
            scratch.at[pl.ds(i * rows, rows)],
            out_ref.at[pl.ds(i * rows, rows)],
            out_sems.at[i],
        )

    for i in range(_N_CHUNKS):
        in_copy(i).start()
    for i in range(_N_CHUNKS):
        in_copy(i).wait()
        out_copy(i).start()
    for i in range(_N_CHUNKS):
        out_copy(i).wait()


def kernel(x, table):
    seq_len = x.shape[1]
    d_model = table.shape[1]
    out = pl.pallas_call(
        _dma_copy,
        in_specs=[pl.BlockSpec(memory_space=pl.ANY)],
        out_specs=pl.BlockSpec(memory_space=pl.ANY),
        out_shape=jax.ShapeDtypeStruct((seq_len, d_model), table.dtype),
        scratch_shapes=[
            pltpu.VMEM((seq_len, d_model), table.dtype),
            pltpu.SemaphoreType.DMA((_N_CHUNKS,)),
            pltpu.SemaphoreType.DMA((_N_CHUNKS,)),
        ],
    )(table)
    return out[None, :, :]
